# Initial kernel scaffold; baseline (speedup 1.0000x reference)
import jax
import jax.numpy as jnp
from jax.experimental import pallas as pl


def kernel(x):
    raise NotImplementedError("write your pallas kernel here")


# speedup vs baseline: 3.3309x; 3.3309x over previous
import jax
import jax.numpy as jnp
from jax import lax
from jax.experimental import pallas as pl

_K = 64
_ROWS_PER_BLOCK = 8


def _topk_mask_body(x_ref, o_ref):
    x = x_ref[...]
    r = x.shape[0]

    raw = lax.bitcast_convert_type(x, jnp.int32)
    ikey = jnp.where(raw >= 0, raw, raw ^ jnp.int32(0x7FFFFFFF))

    prefix = jnp.full((r, 1), -2147483648, dtype=jnp.int32)
    for b in range(31, -1, -1):
        bitval = jnp.int32(-2147483648) if b == 31 else jnp.int32(1 << b)
        trial = prefix ^ bitval
        cnt = jnp.sum((ikey >= trial).astype(jnp.int32), axis=1, keepdims=True)
        prefix = jnp.where(cnt >= _K, trial, prefix)
    t = prefix

    gt = ikey > t
    eq = ikey == t
    cnt_gt = jnp.sum(gt.astype(jnp.int32), axis=1, keepdims=True)
    need = _K - cnt_gt

    idx = lax.broadcasted_iota(jnp.int32, x.shape, 1)
    ipref = jnp.zeros((r, 1), dtype=jnp.int32)
    for b in range(14, -1, -1):
        trial = ipref + jnp.int32(1 << b)
        c = jnp.sum((eq & (idx < trial)).astype(jnp.int32), axis=1,
                    keepdims=True)
        ipref = jnp.where(c < need, trial, ipref)
    cutoff = ipref

    mask = gt | (eq & (idx <= cutoff))
    o_ref[...] = jnp.where(mask, jnp.maximum(x, 0.0), 0.0)


def kernel(x):
    rows, n = x.shape
    rb = _ROWS_PER_BLOCK
    return pl.pallas_call(
        _topk_mask_body,
        grid=(rows // rb,),
        in_specs=[pl.BlockSpec((rb, n), lambda i: (i, 0))],
        out_specs=pl.BlockSpec((rb, n), lambda i: (i, 0)),
        out_shape=jax.ShapeDtypeStruct(x.shape, x.dtype),
    )(x)


# speedup vs baseline: 4.1790x; 1.2546x over previous
import jax
import jax.numpy as jnp
from jax import lax
from jax.experimental import pallas as pl

_K = 64
_ROWS_PER_BLOCK = 8
_INT_MIN32 = -2147483648


def _bitval(b):
    return jnp.int32(_INT_MIN32) if b == 31 else jnp.int32(1 << b)


def _topk_mask_body(x_ref, o_ref):
    x = x_ref[...]
    r, n = x.shape

    raw = lax.bitcast_convert_type(x, jnp.int32)
    ikey = jnp.where(raw >= 0, raw, raw ^ jnp.int32(0x7FFFFFFF))

    m = jnp.max(ikey.reshape(r, n // 128, 128), axis=1)
    u = jnp.max(m, axis=1, keepdims=True)

    lp = jnp.full((r, 1), _INT_MIN32, jnp.int32)
    for b in range(31, -1, -1):
        tr = lp ^ _bitval(b)
        c = jnp.sum((m >= tr).astype(jnp.int32), axis=1, keepdims=True)
        lp = jnp.where(c >= _K, tr, lp)
    lo = lp

    prefix = jnp.full((r, 1), _INT_MIN32, jnp.int32)
    for b in range(31, -1, -1):
        trial = prefix ^ _bitval(b)
        needs = (trial > lo) & (trial <= u)
        cnt = lax.cond(
            jnp.any(needs),
            lambda t=trial: jnp.sum((ikey >= t).astype(jnp.int32), axis=1,
                                    keepdims=True),
            lambda: jnp.zeros((r, 1), jnp.int32),
        )
        take = (trial <= lo) | (needs & (cnt >= _K))
        prefix = jnp.where(take, trial, prefix)
    t = prefix

    cnt_ge = jnp.sum((ikey >= t).astype(jnp.int32), axis=1, keepdims=True)
    ties_any = jnp.any(cnt_ge > _K)

    @pl.when(jnp.logical_not(ties_any))
    def _no_ties():
        o_ref[...] = jnp.where(ikey >= t, jnp.maximum(x, 0.0), 0.0)

    @pl.when(ties_any)
    def _with_ties():
        gt = ikey > t
        eq = ikey == t
        cnt_gt = jnp.sum(gt.astype(jnp.int32), axis=1, keepdims=True)
        need = _K - cnt_gt
        idx = lax.broadcasted_iota(jnp.int32, x.shape, 1)
        ipref = jnp.zeros((r, 1), dtype=jnp.int32)
        for b in range(14, -1, -1):
            tr = ipref + jnp.int32(1 << b)
            c = jnp.sum((eq & (idx < tr)).astype(jnp.int32), axis=1,
                        keepdims=True)
            ipref = jnp.where(c < need, tr, ipref)
        mask = gt | (eq & (idx <= ipref))
        o_ref[...] = jnp.where(mask, jnp.maximum(x, 0.0), 0.0)


def kernel(x):
    rows, n = x.shape
    rb = _ROWS_PER_BLOCK
    return pl.pallas_call(
        _topk_mask_body,
        grid=(rows // rb,),
        in_specs=[pl.BlockSpec((rb, n), lambda i: (i, 0))],
        out_specs=pl.BlockSpec((rb, n), lambda i: (i, 0)),
        out_shape=jax.ShapeDtypeStruct(x.shape, x.dtype),
    )(x)


# speedup vs baseline: 6.9068x; 1.6527x over previous
import jax
import jax.numpy as jnp
from jax import lax
from jax.experimental import pallas as pl

_K = 64
_ROWS_PER_BLOCK = 8
_INT_MIN32 = -2147483648


def _bitval(b):
    return jnp.int32(_INT_MIN32) if b == 31 else jnp.int32(1 << b)


def _topk_mask_body(x_ref, o_ref):
    x = x_ref[...]
    r, n = x.shape

    raw = lax.bitcast_convert_type(x, jnp.int32)
    ikey = jnp.where(raw >= 0, raw, raw ^ jnp.int32(0x7FFFFFFF))

    def count_ge(t):
        return jnp.sum((ikey >= t).astype(jnp.int32), axis=1, keepdims=True)

    m = jnp.max(ikey.reshape(r, n // 128, 128), axis=1)
    u = jnp.max(m, axis=1, keepdims=True)

    lp = jnp.full((r, 1), _INT_MIN32, jnp.int32)
    for b in range(31, -1, -1):
        tr = lp ^ _bitval(b)
        c = jnp.sum((m >= tr).astype(jnp.int32), axis=1, keepdims=True)
        lp = jnp.where(c >= _K, tr, lp)

    lo0 = lp
    cl0 = count_ge(lo0)
    hi0 = u + 1
    ch0 = jnp.zeros((r, 1), jnp.int32)

    def active(lo, hi, cl):
        return (cl > _K) & (hi - 1 > lo)

    def loop_cond(state):
        it, lo, hi, cl, ch = state
        return jnp.any(active(lo, hi, cl))

    def loop_body(state):
        it, lo, hi, cl, ch = state
        act = active(lo, hi, cl)
        wf = lo.astype(jnp.float32) * (-1.0) + hi.astype(jnp.float32)
        frac = (cl - _K).astype(jnp.float32) / jnp.maximum(
            (cl - ch).astype(jnp.float32), 1.0)
        stepf = jnp.clip(wf * frac, 1.0, jnp.maximum(wf - 1.0, 1.0))
        mid_i = lo + stepf.astype(jnp.int32)
        mid_b = (lo & hi) + ((lo ^ hi) >> 1)
        mid = jnp.where(it % 2 == 0, mid_i, mid_b)
        mid = jnp.maximum(lo + 1, jnp.minimum(mid, hi - 1))
        c = count_ge(mid)
        up = c >= _K
        lo = jnp.where(act & up, mid, lo)
        cl = jnp.where(act & up, c, cl)
        hi = jnp.where(act & ~up, mid, hi)
        ch = jnp.where(act & ~up, c, ch)
        return it + 1, lo, hi, cl, ch

    _, lo, hi, cl, ch = lax.while_loop(
        loop_cond, loop_body,
        (jnp.int32(0), lo0, hi0, cl0, ch0))

    t = lo
    ties_any = jnp.any(cl > _K)

    @pl.when(jnp.logical_not(ties_any))
    def _no_ties():
        o_ref[...] = jnp.where(ikey >= t, jnp.maximum(x, 0.0), 0.0)

    @pl.when(ties_any)
    def _with_ties():
        gt = ikey > t
        eq = ikey == t
        cnt_gt = jnp.sum(gt.astype(jnp.int32), axis=1, keepdims=True)
        need = _K - cnt_gt
        idx = lax.broadcasted_iota(jnp.int32, x.shape, 1)
        ipref = jnp.zeros((r, 1), dtype=jnp.int32)
        for b in range(14, -1, -1):
            tr = ipref + jnp.int32(1 << b)
            c = jnp.sum((eq & (idx < tr)).astype(jnp.int32), axis=1,
                        keepdims=True)
            ipref = jnp.where(c < need, tr, ipref)
        mask = gt | (eq & (idx <= ipref))
        o_ref[...] = jnp.where(mask, jnp.maximum(x, 0.0), 0.0)


def kernel(x):
    rows, n = x.shape
    rb = _ROWS_PER_BLOCK
    return pl.pallas_call(
        _topk_mask_body,
        grid=(rows // rb,),
        in_specs=[pl.BlockSpec((rb, n), lambda i: (i, 0))],
        out_specs=pl.BlockSpec((rb, n), lambda i: (i, 0)),
        out_shape=jax.ShapeDtypeStruct(x.shape, x.dtype),
    )(x)


# speedup vs baseline: 7.4635x; 1.0806x over previous
import jax
import jax.numpy as jnp
from jax import lax
from jax.experimental import pallas as pl

_K = 64
_ROWS_PER_BLOCK = 8
_INT_MIN32 = -2147483648


def _bitval(b):
    return jnp.int32(_INT_MIN32) if b == 31 else jnp.int32(1 << b)


def _topk_mask_body(x_ref, o_ref):
    x = x_ref[...]
    r, n = x.shape

    raw = lax.bitcast_convert_type(x, jnp.int32)
    ikey = jnp.where(raw >= 0, raw, raw ^ jnp.int32(0x7FFFFFFF))

    def count_ge(t):
        return jnp.sum((ikey >= t).astype(jnp.int32), axis=1, keepdims=True)

    m = jnp.max(ikey.reshape(r, n // 512, 512), axis=1)
    u = jnp.max(m, axis=1, keepdims=True)

    lp = jnp.full((r, 1), _INT_MIN32, jnp.int32)
    for b in range(31, -1, -1):
        tr = lp ^ _bitval(b)
        c = jnp.sum((m >= tr).astype(jnp.int32), axis=1, keepdims=True)
        lp = jnp.where(c >= _K, tr, lp)

    lo0 = lp
    cl0 = count_ge(lo0)
    hi0 = u + 1
    ch0 = jnp.zeros((r, 1), jnp.int32)

    def active(lo, hi, cl):
        return (cl > _K) & (hi - 1 > lo)

    def loop_cond(state):
        it, lo, hi, cl, ch = state
        return jnp.any(active(lo, hi, cl))

    def loop_body(state):
        it, lo, hi, cl, ch = state
        act = active(lo, hi, cl)
        wf = lo.astype(jnp.float32) * (-1.0) + hi.astype(jnp.float32)
        frac = (cl - _K).astype(jnp.float32) / jnp.maximum(
            (cl - ch).astype(jnp.float32), 1.0)
        stepf = jnp.clip(wf * frac, 1.0, jnp.maximum(wf - 1.0, 1.0))
        mid_i = lo + stepf.astype(jnp.int32)
        mid_b = (lo & hi) + ((lo ^ hi) >> 1)
        mid = jnp.where(it % 2 == 0, mid_i, mid_b)
        mid = jnp.maximum(lo + 1, jnp.minimum(mid, hi - 1))
        c = count_ge(mid)
        up = c >= _K
        lo = jnp.where(act & up, mid, lo)
        cl = jnp.where(act & up, c, cl)
        hi = jnp.where(act & ~up, mid, hi)
        ch = jnp.where(act & ~up, c, ch)
        return it + 1, lo, hi, cl, ch

    _, lo, hi, cl, ch = lax.while_loop(
        loop_cond, loop_body,
        (jnp.int32(0), lo0, hi0, cl0, ch0))

    t = lo
    ties_any = jnp.any(cl > _K)

    @pl.when(jnp.logical_not(ties_any))
    def _no_ties():
        o_ref[...] = jnp.where(ikey >= t, jnp.maximum(x, 0.0), 0.0)

    @pl.when(ties_any)
    def _with_ties():
        gt = ikey > t
        eq = ikey == t
        cnt_gt = jnp.sum(gt.astype(jnp.int32), axis=1, keepdims=True)
        need = _K - cnt_gt
        idx = lax.broadcasted_iota(jnp.int32, x.shape, 1)
        ipref = jnp.zeros((r, 1), dtype=jnp.int32)
        for b in range(14, -1, -1):
            tr = ipref + jnp.int32(1 << b)
            c = jnp.sum((eq & (idx < tr)).astype(jnp.int32), axis=1,
                        keepdims=True)
            ipref = jnp.where(c < need, tr, ipref)
        mask = gt | (eq & (idx <= ipref))
        o_ref[...] = jnp.where(mask, jnp.maximum(x, 0.0), 0.0)


def kernel(x):
    rows, n = x.shape
    rb = _ROWS_PER_BLOCK
    return pl.pallas_call(
        _topk_mask_body,
        grid=(rows // rb,),
        in_specs=[pl.BlockSpec((rb, n), lambda i: (i, 0))],
        out_specs=pl.BlockSpec((rb, n), lambda i: (i, 0)),
        out_shape=jax.ShapeDtypeStruct(x.shape, x.dtype),
    )(x)
